# Initial kernel scaffold; baseline (speedup 1.0000x reference)
import jax
import jax.numpy as jnp
from jax.experimental import pallas as pl


def kernel(hidden_states, weight, e_score_correction_bias):
    raise NotImplementedError("write your pallas kernel here")


# speedup vs baseline: 1.7431x; 1.7431x over previous
import jax
import jax.numpy as jnp
from jax.experimental import pallas as pl

TOP_K = 8
N_EXPERTS = 64
N_GROUP = 8
PER_GROUP = N_EXPERTS // N_GROUP
TOPK_GROUP = 4
SCALE = 2.5

TB = 512
NEG = float("-inf")


def _gate_body(x_ref, w_ref, b_ref, idx_ref, wgt_ref):
    x = x_ref[...]
    w = w_ref[...]
    logits = jax.lax.dot_general(
        x, w, (((1,), (1,)), ((), ())),
        preferred_element_type=jnp.float32)
    scores = jax.nn.sigmoid(logits)
    sfc = scores + b_ref[...]

    lane = jax.lax.broadcasted_iota(jnp.int32, (TB, N_EXPERTS), 1)
    gid = lane // PER_GROUP

    gs = jnp.zeros_like(sfc)
    for g in range(N_GROUP):
        gm = gid == g
        v = jnp.where(gm, sfc, NEG)
        m1 = jnp.max(v, axis=-1, keepdims=True)
        am1 = jnp.min(jnp.where(v == m1, lane, N_EXPERTS), axis=-1, keepdims=True)
        m2 = jnp.max(jnp.where(lane == am1, NEG, v), axis=-1, keepdims=True)
        gs = jnp.where(gm, m1 + m2, gs)

    emask = jnp.zeros(sfc.shape, jnp.bool_)
    for _ in range(TOPK_GROUP):
        m = jnp.max(gs, axis=-1, keepdims=True)
        am = jnp.min(jnp.where(gs == m, lane, N_EXPERTS), axis=-1, keepdims=True)
        sel = gid == (am // PER_GROUP)
        emask = jnp.logical_or(emask, sel)
        gs = jnp.where(sel, NEG, gs)

    tmp = jnp.where(emask, sfc, NEG)
    idx_cols = []
    wgt_cols = []
    for _ in range(TOP_K):
        m = jnp.max(tmp, axis=-1, keepdims=True)
        am = jnp.min(jnp.where(tmp == m, lane, N_EXPERTS), axis=-1, keepdims=True)
        hit = lane == am
        wgt_cols.append(jnp.max(jnp.where(hit, scores, NEG), axis=-1, keepdims=True))
        idx_cols.append(am)
        tmp = jnp.where(hit, NEG, tmp)

    idx = jnp.concatenate(idx_cols, axis=1)
    wgt = jnp.concatenate(wgt_cols, axis=1)
    denom = jnp.sum(wgt, axis=-1, keepdims=True) + 1e-20
    wgt = wgt / denom * SCALE
    idx_ref[...] = idx
    wgt_ref[...] = wgt


def kernel(hidden_states, weight, e_score_correction_bias):
    bsz, seq_len, h = hidden_states.shape
    t = bsz * seq_len
    hs = hidden_states.reshape(t, h)
    bias = e_score_correction_bias.reshape(1, N_EXPERTS)
    grid = (t // TB,)
    idx, wgt = pl.pallas_call(
        _gate_body,
        grid=grid,
        in_specs=[
            pl.BlockSpec((TB, h), lambda i: (i, 0)),
            pl.BlockSpec((N_EXPERTS, h), lambda i: (0, 0)),
            pl.BlockSpec((1, N_EXPERTS), lambda i: (0, 0)),
        ],
        out_specs=[
            pl.BlockSpec((TB, TOP_K), lambda i: (i, 0)),
            pl.BlockSpec((TB, TOP_K), lambda i: (i, 0)),
        ],
        out_shape=[
            jax.ShapeDtypeStruct((t, TOP_K), jnp.int32),
            jax.ShapeDtypeStruct((t, TOP_K), jnp.float32),
        ],
    )(hs, weight, bias)
    return idx, wgt


# speedup vs baseline: 4.4667x; 2.5624x over previous
import jax
import jax.numpy as jnp
from jax.experimental import pallas as pl

TOP_K = 8
N_EXPERTS = 64
N_GROUP = 8
PER_GROUP = N_EXPERTS // N_GROUP
TOPK_GROUP = 4
SCALE = 2.5

TB = 512
NEG = float("-inf")


def _gate_body(x_ref, w_ref, b_ref, idx_ref, wgt_ref):
    x = x_ref[...]
    w = w_ref[...]
    logits = jax.lax.dot_general(
        x, w, (((1,), (1,)), ((), ())),
        preferred_element_type=jnp.float32)
    lt = logits.T
    sig = jax.nn.sigmoid(lt)
    sfc = sig + b_ref[...]

    g3 = sfc.reshape(N_GROUP, PER_GROUP, TB)
    m1 = jnp.max(g3, axis=1)
    eq1 = g3 == m1[:, None, :]
    cnt = jnp.sum(eq1.astype(jnp.float32), axis=1)
    m2 = jnp.max(jnp.where(eq1, NEG, g3), axis=1)
    gs = m1 + jnp.where(cnt > 1.5, m1, m2)

    giota = jax.lax.broadcasted_iota(jnp.int32, (N_GROUP, TB), 0)
    gmask = jnp.zeros((N_GROUP, TB), jnp.bool_)
    for _ in range(TOPK_GROUP):
        m = jnp.max(gs, axis=0, keepdims=True)
        am = jnp.min(jnp.where(gs == m, giota, N_GROUP), axis=0, keepdims=True)
        sel = giota == am
        gmask = jnp.logical_or(gmask, sel)
        gs = jnp.where(sel, NEG, gs)
    emask = jnp.broadcast_to(
        gmask[:, None, :], (N_GROUP, PER_GROUP, TB)).reshape(N_EXPERTS, TB)

    eiota = jax.lax.broadcasted_iota(jnp.int32, (N_EXPERTS, TB), 0)
    tmp = jnp.where(emask, sfc, NEG)
    idx_rows = []
    wgt_rows = []
    for _ in range(TOP_K):
        m = jnp.max(tmp, axis=0, keepdims=True)
        am = jnp.min(jnp.where(tmp == m, eiota, N_EXPERTS), axis=0, keepdims=True)
        hit = eiota == am
        wgt_rows.append(jnp.max(jnp.where(hit, sig, NEG), axis=0, keepdims=True))
        idx_rows.append(am)
        tmp = jnp.where(hit, NEG, tmp)

    idx_t = jnp.concatenate(idx_rows, axis=0)
    wgt_t = jnp.concatenate(wgt_rows, axis=0)
    denom = jnp.sum(wgt_t, axis=0, keepdims=True) + 1e-20
    wgt_t = wgt_t / denom * SCALE
    idx_ref[...] = idx_t.T
    wgt_ref[...] = wgt_t.T


def kernel(hidden_states, weight, e_score_correction_bias):
    bsz, seq_len, h = hidden_states.shape
    t = bsz * seq_len
    hs = hidden_states.reshape(t, h)
    bias = e_score_correction_bias.reshape(N_EXPERTS, 1)
    grid = (t // TB,)
    idx, wgt = pl.pallas_call(
        _gate_body,
        grid=grid,
        in_specs=[
            pl.BlockSpec((TB, h), lambda i: (i, 0)),
            pl.BlockSpec((N_EXPERTS, h), lambda i: (0, 0)),
            pl.BlockSpec((N_EXPERTS, 1), lambda i: (0, 0)),
        ],
        out_specs=[
            pl.BlockSpec((TB, TOP_K), lambda i: (i, 0)),
            pl.BlockSpec((TB, TOP_K), lambda i: (i, 0)),
        ],
        out_shape=[
            jax.ShapeDtypeStruct((t, TOP_K), jnp.int32),
            jax.ShapeDtypeStruct((t, TOP_K), jnp.float32),
        ],
    )(hs, weight, bias)
    return idx, wgt
